# baseline (speedup 1.0000x reference)
import functools

import jax
import jax.numpy as jnp
from jax.experimental import pallas as pl
from jax.experimental.pallas import tpu as pltpu


def _round_up(x, m):
    return ((x + m - 1) // m) * m


def _affine_kernel(x_ref, w_ref, b_ref, o_ref, x_bf, w_bf, *, tn):
    j = pl.program_id(1)

    @pl.when(j == 0)
    def _():
        x_bf[...] = x_ref[...].astype(jnp.bfloat16)
        w_bf[...] = w_ref[...].astype(jnp.bfloat16)

    o_ref[...] = (
        jnp.dot(x_bf[...], w_bf[:, pl.ds(j * tn, tn)],
                preferred_element_type=jnp.float32)
        + b_ref[...]
    )


@jax.jit
def _affine(x, w, bias):
    batch, in_dim = x.shape
    _, n = w.shape

    tm = 1024
    tn = n // 2
    m_pad = _round_up(batch, tm)
    x_p = x if m_pad == batch else jnp.pad(x, ((0, m_pad - batch), (0, 0)))

    out = pl.pallas_call(
        functools.partial(_affine_kernel, tn=tn),
        out_shape=jax.ShapeDtypeStruct((m_pad, n), jnp.float32),
        grid=(m_pad // tm, 2),
        in_specs=[
            pl.BlockSpec((tm, in_dim), lambda i, j: (i, 0)),
            pl.BlockSpec((in_dim, n), lambda i, j: (0, 0)),
            pl.BlockSpec((1, tn), lambda i, j: (0, j)),
        ],
        out_specs=pl.BlockSpec((tm, tn), lambda i, j: (i, j)),
        scratch_shapes=[
            pltpu.VMEM((tm, in_dim), jnp.bfloat16),
            pltpu.VMEM((in_dim, n), jnp.bfloat16),
        ],
        compiler_params=pltpu.CompilerParams(
            dimension_semantics=("parallel", "arbitrary"),
            vmem_limit_bytes=56 * 1024 * 1024,
        ),
    )(x_p, w, bias)

    return out[:batch] if m_pad != batch else out


def kernel(x, wt_padded, bias_padded):
    return _affine(x, wt_padded, bias_padded)


# speedup vs baseline: 1.5461x; 1.5461x over previous
import jax
import jax.numpy as jnp
from jax.experimental import pallas as pl
from jax.experimental.pallas import tpu as pltpu


def _round_up(x, m):
    return ((x + m - 1) // m) * m


def _affine_kernel(x_ref, w_ref, b_ref, o_ref, w_bf):
    i = pl.program_id(0)

    @pl.when(jax.lax.rem(i, pl.num_programs(0) // 2) == 0)
    def _():
        w_bf[...] = w_ref[...].astype(jnp.bfloat16)

    xb = x_ref[...].astype(jnp.bfloat16)
    o_ref[...] = (
        jnp.dot(xb, w_bf[...], preferred_element_type=jnp.float32)
        + b_ref[...]
    )


@jax.jit
def _affine(x, w, bias):
    batch, in_dim = x.shape
    _, n = w.shape

    tm = 1024
    m_pad = _round_up(batch, tm)
    x_p = x if m_pad == batch else jnp.pad(x, ((0, m_pad - batch), (0, 0)))

    out = pl.pallas_call(
        _affine_kernel,
        out_shape=jax.ShapeDtypeStruct((m_pad, n), jnp.float32),
        grid=(m_pad // tm,),
        in_specs=[
            pl.BlockSpec((tm, in_dim), lambda i: (i, 0)),
            pl.BlockSpec((in_dim, n), lambda i: (0, 0)),
            pl.BlockSpec((1, n), lambda i: (0, 0)),
        ],
        out_specs=pl.BlockSpec((tm, n), lambda i: (i, 0)),
        scratch_shapes=[pltpu.VMEM((in_dim, n), jnp.bfloat16)],
        compiler_params=pltpu.CompilerParams(
            dimension_semantics=("parallel",),
            vmem_limit_bytes=56 * 1024 * 1024,
        ),
    )(x_p, w, bias)

    return out[:batch] if m_pad != batch else out


def kernel(x, wt_padded, bias_padded):
    return _affine(x, wt_padded, bias_padded)


# speedup vs baseline: 1.5743x; 1.0182x over previous
import jax
import jax.numpy as jnp
from jax.experimental import pallas as pl
from jax.experimental.pallas import tpu as pltpu


def _round_up(x, m):
    return ((x + m - 1) // m) * m


def _affine_kernel(x_ref, w_ref, b_ref, o_ref):
    xb = x_ref[...].astype(jnp.bfloat16)
    wb = w_ref[...].astype(jnp.bfloat16)
    o_ref[...] = (
        jnp.dot(xb, wb, preferred_element_type=jnp.float32)
        + b_ref[...]
    )


@jax.jit
def _affine(x, w, bias):
    batch, in_dim = x.shape
    _, n = w.shape

    tm = 1024
    m_pad = _round_up(batch, tm)
    x_p = x if m_pad == batch else jnp.pad(x, ((0, m_pad - batch), (0, 0)))

    out = pl.pallas_call(
        _affine_kernel,
        out_shape=jax.ShapeDtypeStruct((m_pad, n), jnp.float32),
        grid=(m_pad // tm,),
        in_specs=[
            pl.BlockSpec((tm, in_dim), lambda i: (i, 0)),
            pl.BlockSpec((in_dim, n), lambda i: (0, 0)),
            pl.BlockSpec((1, n), lambda i: (0, 0)),
        ],
        out_specs=pl.BlockSpec((tm, n), lambda i: (i, 0)),
        compiler_params=pltpu.CompilerParams(
            dimension_semantics=("parallel",),
            vmem_limit_bytes=56 * 1024 * 1024,
        ),
    )(x_p, w, bias)

    return out[:batch] if m_pad != batch else out


def kernel(x, wt_padded, bias_padded):
    return _affine(x, wt_padded, bias_padded)
